# baseline (speedup 1.0000x reference)
import functools

import jax
import jax.numpy as jnp
from jax import lax
from jax.experimental import pallas as pl
from jax.experimental.pallas import tpu as pltpu
from jax.experimental.pallas import tpu_sc as plsc

EMBED = 1024
B, T = 1024, 50
TP = 64


def _proj_body(emb_ref, pos_ref, w_ref, b_ref, out_ref):
    a = emb_ref[...] + pos_ref[...]
    out_ref[...] = (
        lax.dot_general(
            a, w_ref[...],
            dimension_numbers=(((1,), (1,)), ((), ())),
            precision=lax.Precision.HIGHEST,
            preferred_element_type=jnp.float32,
        )
        + b_ref[...]
    )


def _project_table(emb_table, pos_table, W, b2d):
    return pl.pallas_call(
        _proj_body,
        out_shape=jax.ShapeDtypeStruct((EMBED, EMBED), jnp.float32),
    )(emb_table, pos_table, W, b2d)


_INFO = plsc.get_sparse_core_info()
_NC, _NS = _INFO.num_cores, _INFO.num_subcores
_NW = _NC * _NS
_SLABS_W = B // _NW


def _gather_body(table_hbm, idx_hbm, out_hbm, idx_all, rows_v, sem):
    wid = lax.axis_index("s") * _NC + lax.axis_index("c")
    pltpu.sync_copy(
        idx_hbm.at[pl.ds(wid * (_SLABS_W * TP), _SLABS_W * TP)], idx_all
    )
    base = wid * _SLABS_W

    def body(c, carry):
        pltpu.async_copy(
            table_hbm.at[idx_all.at[pl.ds(c * TP, TP)]], rows_v, sem
        ).wait()
        pltpu.sync_copy(rows_v.at[pl.ds(0, T)], out_hbm.at[base + c])
        return carry

    lax.fori_loop(0, _SLABS_W, body, 0)


_gather = functools.partial(
    pl.kernel,
    out_type=jax.ShapeDtypeStruct((B, T, 8, 128), jnp.float32),
    mesh=plsc.VectorSubcoreMesh(core_axis_name="c", subcore_axis_name="s"),
    scratch_types=[
        pltpu.VMEM((_SLABS_W * TP,), jnp.int32),
        pltpu.VMEM((TP, 8, 128), jnp.float32),
        pltpu.SemaphoreType.DMA,
    ],
)(_gather_body)


def kernel(x, emb_table, pos_table, W, b):
    proj = _project_table(emb_table, pos_table, W, b.reshape(1, EMBED))
    xp = jnp.pad(x, ((0, 0), (0, TP - T))).reshape(-1)
    out4 = _gather(proj.reshape(EMBED, 8, 128), xp)
    return out4.reshape(B, T, EMBED)


# speedup vs baseline: 1.2138x; 1.2138x over previous
import functools

import jax
import jax.numpy as jnp
from jax import lax
from jax.experimental import pallas as pl
from jax.experimental.pallas import tpu as pltpu
from jax.experimental.pallas import tpu_sc as plsc

EMBED = 1024
B, T = 1024, 50
TT = 7
NROW = B * TT * 8 * 8


def _proj_body(emb_ref, pos_ref, w_ref, b_ref, out_ref):
    a = emb_ref[...] + pos_ref[...]
    out_ref[...] = (
        lax.dot_general(
            a, w_ref[...],
            dimension_numbers=(((1,), (1,)), ((), ())),
            precision=lax.Precision.HIGHEST,
            preferred_element_type=jnp.float32,
        )
        + b_ref[...]
    )


def _project_table(emb_table, pos_table, W, b2d):
    return pl.pallas_call(
        _proj_body,
        out_shape=jax.ShapeDtypeStruct((EMBED, EMBED), jnp.float32),
    )(emb_table, pos_table, W, b2d)


_INFO = plsc.get_sparse_core_info()
_NC, _NS = _INFO.num_cores, _INFO.num_subcores
_NW = _NC * _NS
_PER_W = NROW // _NW
_CH = 256
_NCH = _PER_W // _CH
_NPAIR = _NCH // 2


def _gather_body(table_hbm, idx_hbm, out_hbm,
                 idx_all, ra, rb, gsa, gsb, ssa, ssb):
    wid = lax.axis_index("s") * _NC + lax.axis_index("c")
    ibase = wid * _PER_W
    pltpu.sync_copy(idx_hbm.at[pl.ds(ibase, _PER_W)], idx_all)

    def gstart(buf, sem, c):
        off = c * _CH
        pltpu.async_copy(
            table_hbm.at[idx_all.at[pl.ds(off, 128)]],
            buf.at[pl.ds(0, 128)], sem)
        pltpu.async_copy(
            table_hbm.at[idx_all.at[pl.ds(off + 128, 128)]],
            buf.at[pl.ds(128, 128)], sem)

    def gwait(buf, sem):
        pltpu.make_async_copy(table_hbm.at[pl.ds(0, _CH)], buf, sem).wait()

    def wstart(buf, sem, c):
        pltpu.async_copy(buf, out_hbm.at[pl.ds(ibase + c * _CH, _CH)], sem)

    def wwait(buf, sem):
        pltpu.make_async_copy(
            buf, out_hbm.at[pl.ds(ibase, _CH)], sem).wait()

    gstart(ra, gsa, 0)

    def step(k, carry):
        @pl.when(k > 0)
        def _():
            wwait(rb, ssb)

        gstart(rb, gsb, 2 * k + 1)
        gwait(ra, gsa)
        wstart(ra, ssa, 2 * k)

        @pl.when(k < _NPAIR - 1)
        def _():
            wwait(ra, ssa)
            gstart(ra, gsa, 2 * k + 2)

        gwait(rb, gsb)
        wstart(rb, ssb, 2 * k + 1)
        return carry

    lax.fori_loop(0, _NPAIR, step, 0)
    wwait(ra, ssa)
    wwait(rb, ssb)


_tile_gather = functools.partial(
    pl.kernel,
    out_type=jax.ShapeDtypeStruct((NROW, 128), jnp.float32),
    mesh=plsc.VectorSubcoreMesh(core_axis_name="c", subcore_axis_name="s"),
    scratch_types=[
        pltpu.VMEM((_PER_W,), jnp.int32),
        pltpu.VMEM((_CH, 128), jnp.float32),
        pltpu.VMEM((_CH, 128), jnp.float32),
        pltpu.SemaphoreType.DMA,
        pltpu.SemaphoreType.DMA,
        pltpu.SemaphoreType.DMA,
        pltpu.SemaphoreType.DMA,
    ],
)(_gather_body)


_BB = 4


def _asm_body(g_ref, out_ref):
    for bl in range(_BB):
        for tau in range(TT):
            for s in range(8):
                src = g_ref[pl.ds((bl * TT * 8 + tau * 8 + s) * 8, 8), :]
                if tau < TT - 1:
                    out_ref[bl, 8 * tau:8 * tau + 8, 128 * s:128 * (s + 1)] = src
                else:
                    out_ref[bl, 48:T, 128 * s:128 * (s + 1)] = src[0:2, :]


def _assemble(g):
    return pl.pallas_call(
        _asm_body,
        grid=(B // _BB,),
        in_specs=[pl.BlockSpec((_BB * TT * 64, 128), lambda i: (i, 0))],
        out_specs=pl.BlockSpec((_BB, T, EMBED), lambda i: (i, 0, 0)),
        out_shape=jax.ShapeDtypeStruct((B, T, EMBED), jnp.float32),
    )(g)


def kernel(x, emb_table, pos_table, W, b):
    proj = _project_table(emb_table, pos_table, W, b.reshape(1, EMBED))
    xp = jnp.pad(x, ((0, 0), (0, TT * 8 - T))).reshape(B, TT, 8)
    idx = (xp[:, :, None, :] * 8
           + jnp.arange(8, dtype=jnp.int32)[None, None, :, None])
    g = _tile_gather(proj.reshape(EMBED * 8, 128), idx.reshape(-1))
    return _assemble(g)


# speedup vs baseline: 2.6303x; 2.1670x over previous
import functools

import jax
import jax.numpy as jnp
from jax import lax
from jax.experimental import pallas as pl
from jax.experimental.pallas import tpu as pltpu
from jax.experimental.pallas import tpu_sc as plsc

EMBED = 1024
B, T = 1024, 50
N_TOK = B * T


def _proj_body(emb_ref, pos_ref, w_ref, b_ref, out_ref):
    a = emb_ref[...] + pos_ref[...]
    out_ref[...] = (
        lax.dot_general(
            a, w_ref[...],
            dimension_numbers=(((1,), (1,)), ((), ())),
            precision=lax.Precision.HIGHEST,
            preferred_element_type=jnp.float32,
        )
        + b_ref[...]
    )


def _project_table(emb_table, pos_table, W, b2d):
    return pl.pallas_call(
        _proj_body,
        out_shape=jax.ShapeDtypeStruct((EMBED, EMBED), jnp.float32),
    )(emb_table, pos_table, W, b2d)


_INFO = plsc.get_sparse_core_info()
_NC, _NS = _INFO.num_cores, _INFO.num_subcores
_NW = _NC * _NS
_PER_W = N_TOK // _NW
_CH = 32
_NCH = _PER_W // _CH
_NPAIR = _NCH // 2


def _gather_body(table_hbm, idx_hbm, out_hbm,
                 idx_all, ra, rb, gsa, gsb, ssa, ssb):
    wid = lax.axis_index("s") * _NC + lax.axis_index("c")
    base = wid * _PER_W
    pltpu.sync_copy(idx_hbm.at[pl.ds(base, _PER_W)], idx_all)

    def gstart(buf, sem, c):
        pltpu.async_copy(
            table_hbm.at[idx_all.at[pl.ds(c * _CH, _CH)]], buf, sem)

    def gwait(buf, sem):
        pltpu.make_async_copy(table_hbm.at[pl.ds(0, _CH)], buf, sem).wait()

    def wstart(buf, sem, c):
        pltpu.async_copy(buf, out_hbm.at[pl.ds(base + c * _CH, _CH)], sem)

    def wwait(buf, sem):
        pltpu.make_async_copy(
            buf, out_hbm.at[pl.ds(base, _CH)], sem).wait()

    gstart(ra, gsa, 0)

    def step(k, carry):
        @pl.when(k > 0)
        def _():
            wwait(rb, ssb)

        gstart(rb, gsb, 2 * k + 1)
        gwait(ra, gsa)
        wstart(ra, ssa, 2 * k)

        @pl.when(k < _NPAIR - 1)
        def _():
            wwait(ra, ssa)
            gstart(ra, gsa, 2 * k + 2)

        gwait(rb, gsb)
        wstart(rb, ssb, 2 * k + 1)
        return carry

    lax.fori_loop(0, _NPAIR, step, 0)
    wwait(ra, ssa)
    wwait(rb, ssb)


_gather = functools.partial(
    pl.kernel,
    out_type=jax.ShapeDtypeStruct((N_TOK, EMBED), jnp.float32),
    mesh=plsc.VectorSubcoreMesh(core_axis_name="c", subcore_axis_name="s"),
    scratch_types=[
        pltpu.VMEM((_PER_W,), jnp.int32),
        pltpu.VMEM((_CH, EMBED), jnp.float32),
        pltpu.VMEM((_CH, EMBED), jnp.float32),
        pltpu.SemaphoreType.DMA,
        pltpu.SemaphoreType.DMA,
        pltpu.SemaphoreType.DMA,
        pltpu.SemaphoreType.DMA,
    ],
)(_gather_body)


def kernel(x, emb_table, pos_table, W, b):
    proj = _project_table(emb_table, pos_table, W, b.reshape(1, EMBED))
    out = _gather(proj, x.reshape(-1))
    return out.reshape(B, T, EMBED)
